# baseline (speedup 1.0000x reference)
import functools

import jax
import jax.numpy as jnp
from jax import lax
from jax.experimental import pallas as pl
from jax.experimental.pallas import tpu as pltpu
from jax.experimental.pallas import tpu_sc as plsc

_EMB = 128
_NC, _NS, _L = 2, 16, 16
_NW = _NC * _NS
_CH = 256


def _combo_body(w0_ref, w1_ref, w2_ref, out_ref):
    out_ref[...] = jnp.zeros((64, _EMB), jnp.float32)
    for v in range(60):
        a, b, c = v // 12, (v // 2) % 6, v % 2
        out_ref[v : v + 1, :] = (
            w0_ref[a : a + 1, :] + w1_ref[b : b + 1, :] + w2_ref[c : c + 1, :]
        )


def _sc_body(combo_hbm, x0_hbm, x1_hbm, x2_hbm, out_hbm,
             i0_v, i1_v, i2_v, cidx_v, rows_v, sem, *, n_chunks):
    wid = lax.axis_index("s") * _NC + lax.axis_index("c")
    hi = n_chunks // _NW + 1
    rem = n_chunks % _NW
    nch = jnp.where(wid < rem, hi, hi - 1)

    def body(j, carry):
        base = (wid + _NW * j) * _CH
        pltpu.sync_copy(x0_hbm.at[pl.ds(base, _CH)], i0_v)
        pltpu.sync_copy(x1_hbm.at[pl.ds(base, _CH)], i1_v)
        pltpu.sync_copy(x2_hbm.at[pl.ds(base, _CH)], i2_v)
        for g in range(_CH // _L):
            s = pl.ds(_L * g, _L)
            c = (i0_v[s] * 6 + i1_v[s]) * 2 + i2_v[s]
            cidx_v[g // 8, pl.ds((g % 8) * _L, _L)] = c
        cps = [
            pltpu.async_copy(
                combo_hbm.at[cidx_v.at[t]],
                rows_v.at[pl.ds(128 * t, 128)],
                sem,
            )
            for t in range(_CH // 128)
        ]
        for cp in cps:
            cp.wait()
        pltpu.sync_copy(rows_v, out_hbm.at[pl.ds(base, _CH)])
        return carry

    lax.fori_loop(0, nch, body, 0)


def kernel(inputs, W0, W1, W2):
    E = inputs.shape[0]
    assert E % _CH == 0
    combo = pl.pallas_call(
        _combo_body,
        out_shape=jax.ShapeDtypeStruct((64, _EMB), jnp.float32),
    )(W0, W1, W2)
    xt = inputs.T
    x0, x1, x2 = xt[0], xt[1], xt[2]

    sc = pl.kernel(
        functools.partial(_sc_body, n_chunks=E // _CH),
        out_type=jax.ShapeDtypeStruct((E, _EMB), jnp.float32),
        mesh=plsc.VectorSubcoreMesh(
            core_axis_name="c", subcore_axis_name="s",
            num_cores=_NC, num_subcores=_NS,
        ),
        scratch_types=[
            pltpu.VMEM((_CH,), jnp.int32),
            pltpu.VMEM((_CH,), jnp.int32),
            pltpu.VMEM((_CH,), jnp.int32),
            pltpu.VMEM((_CH // 128, 128), jnp.int32),
            pltpu.VMEM((_CH, _EMB), jnp.float32),
            pltpu.SemaphoreType.DMA,
        ],
    )
    return sc(combo, x0, x1, x2)


# speedup vs baseline: 4.9693x; 4.9693x over previous
import functools

import jax
import jax.numpy as jnp
from jax import lax
from jax.experimental import pallas as pl
from jax.experimental.pallas import tpu as pltpu
from jax.experimental.pallas import tpu_sc as plsc

_EMB = 128
_NC, _NS, _L = 2, 16, 16
_NW = _NC * _NS
_CH = 128
_NSTREAM = 4


def _combo_body(w0_ref, w1_ref, w2_ref, out_ref):
    out_ref[...] = jnp.zeros((64, _EMB), jnp.float32)
    for v in range(60):
        a, b, c = v // 12, (v // 2) % 6, v % 2
        out_ref[v : v + 1, :] = (
            w0_ref[a : a + 1, :] + w1_ref[b : b + 1, :] + w2_ref[c : c + 1, :]
        )


def _sc_body(combo_hbm, x0_hbm, x1_hbm, x2_hbm, out_hbm,
             idx_v, cidx_v, rows_v, isems, gsems, osems, *, n_chunks, n_gen):
    wid = lax.axis_index("s") * _NC + lax.axis_index("c")

    def chunk_id(g, s):
        return wid + _NW * (_NSTREAM * g + s)

    def idx_copies(g, s):
        base = chunk_id(g, s) * _CH
        return [
            pltpu.make_async_copy(
                x_hbm.at[pl.ds(base, _CH)], idx_v.at[s, c], isems[s])
            for c, x_hbm in enumerate((x0_hbm, x1_hbm, x2_hbm))
        ]

    def gather_copy(g, s):
        return pltpu.make_async_copy(
            combo_hbm.at[cidx_v.at[s, 0]], rows_v.at[s], gsems[s])

    def out_copy(g, s):
        base = chunk_id(g, s) * _CH
        return pltpu.make_async_copy(
            rows_v.at[s], out_hbm.at[pl.ds(base, _CH)], osems[s])

    def valid(g, s):
        return chunk_id(g, s) < n_chunks

    for s in range(_NSTREAM):
        @pl.when(valid(0, s))
        def _(s=s):
            for cp in idx_copies(0, s):
                cp.start()

    def body(g, carry):
        for s in range(_NSTREAM):
            @pl.when(jnp.logical_and(g >= 1, valid(g - 1, s)))
            def _(s=s):
                gather_copy(g - 1, s).wait()
                out_copy(g - 1, s).start()
        for s in range(_NSTREAM):
            @pl.when(valid(g, s))
            def _(s=s):
                for cp in idx_copies(g, s):
                    cp.wait()
                for grp in range(_CH // _L):
                    sl = pl.ds(_L * grp, _L)
                    c = (idx_v[s, 0, sl] * 6 + idx_v[s, 1, sl]) * 2 \
                        + idx_v[s, 2, sl]
                    cidx_v[s, 0, sl] = c + wid * 64

                @pl.when(g >= 1)
                def _():
                    out_copy(g - 1, s).wait()

                gather_copy(g, s).start()

            @pl.when(valid(g + 1, s))
            def _(s=s):
                for cp in idx_copies(g + 1, s):
                    cp.start()
        return carry

    lax.fori_loop(0, n_gen + 1, body, 0)

    for s in range(_NSTREAM):
        @pl.when(valid(0, s))
        def _(s=s):
            out_copy(0, s).wait()


def kernel(inputs, W0, W1, W2):
    E = inputs.shape[0]
    assert E % _CH == 0
    n_chunks = E // _CH
    max_m = -(-n_chunks // _NW)
    n_gen = -(-max_m // _NSTREAM)
    combo = pl.pallas_call(
        _combo_body,
        out_shape=jax.ShapeDtypeStruct((64, _EMB), jnp.float32),
    )(W0, W1, W2)
    combo_rep = jnp.tile(combo, (_NW, 1))
    xt = inputs.T
    x0, x1, x2 = xt[0], xt[1], xt[2]

    sc = pl.kernel(
        functools.partial(_sc_body, n_chunks=n_chunks, n_gen=n_gen),
        out_type=jax.ShapeDtypeStruct((E, _EMB), jnp.float32),
        mesh=plsc.VectorSubcoreMesh(
            core_axis_name="c", subcore_axis_name="s",
            num_cores=_NC, num_subcores=_NS,
        ),
        scratch_types=[
            pltpu.VMEM((_NSTREAM, 3, _CH), jnp.int32),
            pltpu.VMEM((_NSTREAM, 1, _CH), jnp.int32),
            pltpu.VMEM((_NSTREAM, _CH, _EMB), jnp.float32),
            [pltpu.SemaphoreType.DMA] * _NSTREAM,
            [pltpu.SemaphoreType.DMA] * _NSTREAM,
            [pltpu.SemaphoreType.DMA] * _NSTREAM,
        ],
    )
    return sc(combo_rep, x0, x1, x2)


# speedup vs baseline: 4.9851x; 1.0032x over previous
import functools

import jax
import jax.numpy as jnp
from jax import lax
from jax.experimental import pallas as pl
from jax.experimental.pallas import tpu as pltpu
from jax.experimental.pallas import tpu_sc as plsc

_EMB = 128
_NC, _NS, _L = 2, 16, 16
_NW = _NC * _NS
_CH = 128
_NSTR = 2


def _combo_body(w0_ref, w1_ref, w2_ref, out_ref):
    out_ref[...] = jnp.zeros((64, _EMB), jnp.float32)
    for v in range(60):
        a, b, c = v // 12, (v // 2) % 6, v % 2
        out_ref[v : v + 1, :] = (
            w0_ref[a : a + 1, :] + w1_ref[b : b + 1, :] + w2_ref[c : c + 1, :]
        )


def _sc_body(combo_hbm, x0_hbm, x1_hbm, x2_hbm, out_hbm,
             idx_v, cidx_v, rows_v, isems, gsems, osems, *, n_chunks, n_gen):
    wid = lax.axis_index("s") * _NC + lax.axis_index("c")

    def chunk_id(g, s):
        return wid + _NW * (_NSTR * g + s)

    def valid(g, s):
        return chunk_id(g, s) < n_chunks

    def idx_copies(g, s, p):
        base = chunk_id(g, s) * _CH
        return [
            pltpu.make_async_copy(
                x_hbm.at[pl.ds(base, _CH)], idx_v.at[s, p, c], isems[2 * s + p])
            for c, x_hbm in enumerate((x0_hbm, x1_hbm, x2_hbm))
        ]

    def gather_copy(g, s, p):
        return pltpu.make_async_copy(
            combo_hbm.at[cidx_v.at[s, p, 0]], rows_v.at[s, p],
            gsems[2 * s + p])

    def out_copy(g, s, p):
        base = chunk_id(g, s) * _CH
        return pltpu.make_async_copy(
            rows_v.at[s, p], out_hbm.at[pl.ds(base, _CH)], osems[s])

    def half_body(g, p):
        for s in range(_NSTR):
            @pl.when(valid(g, s))
            def _(s=s):
                for cp in idx_copies(g, s, p):
                    cp.wait()
                for grp in range(_CH // _L):
                    sl = pl.ds(_L * grp, _L)
                    c = (idx_v[s, p, 0, sl] * 6 + idx_v[s, p, 1, sl]) * 2 \
                        + idx_v[s, p, 2, sl]
                    cidx_v[s, p, 0, sl] = c + wid * 64

                @pl.when(g >= 2)
                def _():
                    out_copy(g - 2, s, p).wait()

                gather_copy(g, s, p).start()

            @pl.when(valid(g + 1, s))
            def _(s=s):
                for cp in idx_copies(g + 1, s, 1 - p):
                    cp.start()
        for s in range(_NSTR):
            @pl.when(jnp.logical_and(g >= 1, valid(g - 1, s)))
            def _(s=s):
                gather_copy(g - 1, s, 1 - p).wait()
                out_copy(g - 1, s, 1 - p).start()

    for s in range(_NSTR):
        @pl.when(valid(0, s))
        def _(s=s):
            for cp in idx_copies(0, s, 0):
                cp.start()

    def body(t, carry):
        half_body(2 * t, 0)
        half_body(2 * t + 1, 1)
        return carry

    lax.fori_loop(0, n_gen // 2 + 1, body, 0)

    for s in range(_NSTR):
        for p in range(2):
            @pl.when(valid(p, s))
            def _(s=s, p=p):
                out_copy(p, s, p).wait()


def kernel(inputs, W0, W1, W2):
    E = inputs.shape[0]
    assert E % _CH == 0
    n_chunks = E // _CH
    max_m = -(-n_chunks // _NW)
    n_gen = -(-max_m // _NSTR)
    combo = pl.pallas_call(
        _combo_body,
        out_shape=jax.ShapeDtypeStruct((64, _EMB), jnp.float32),
    )(W0, W1, W2)
    combo_rep = jnp.tile(combo, (_NW, 1))
    xt = inputs.T
    x0, x1, x2 = xt[0], xt[1], xt[2]

    sc = pl.kernel(
        functools.partial(_sc_body, n_chunks=n_chunks, n_gen=n_gen),
        out_type=jax.ShapeDtypeStruct((E, _EMB), jnp.float32),
        mesh=plsc.VectorSubcoreMesh(
            core_axis_name="c", subcore_axis_name="s",
            num_cores=_NC, num_subcores=_NS,
        ),
        scratch_types=[
            pltpu.VMEM((_NSTR, 2, 3, _CH), jnp.int32),
            pltpu.VMEM((_NSTR, 2, 1, _CH), jnp.int32),
            pltpu.VMEM((_NSTR, 2, _CH, _EMB), jnp.float32),
            [pltpu.SemaphoreType.DMA] * (2 * _NSTR),
            [pltpu.SemaphoreType.DMA] * (2 * _NSTR),
            [pltpu.SemaphoreType.DMA] * _NSTR,
        ],
    )
    return sc(combo_rep, x0, x1, x2)
